# baseline (speedup 1.0000x reference)
import functools

import jax
import jax.numpy as jnp
from jax import lax
from jax.experimental import pallas as pl
from jax.experimental.pallas import tpu as pltpu
from jax.experimental.pallas import tpu_sc as plsc

N = 10000
E = 320000
F_IN = 128
OUT = 128
HW = 64
H1 = 10
G = 64

NP = 10240
NBLK = 1024
EP = 327680
CH = 128
NCH = EP // CH
RPT = NP // 16


def _tck1_body(x_ref, w_ref, atts_ref, attd_ref, h_ref, as_ref, ad_ref):
    xb = x_ref[...]
    W = w_ref[...]
    hb = jnp.dot(xb, W, preferred_element_type=jnp.float32)
    cols_s = []
    cols_d = []
    for h in range(H1):
        h_ref[2 * h] = hb[:, h * OUT:h * OUT + HW]
        h_ref[2 * h + 1] = hb[:, h * OUT + HW:(h + 1) * OUT]
        Wh = W[:, h * OUT:(h + 1) * OUT]
        ats = atts_ref[...][h:h + 1, :]
        atd = attd_ref[...][h:h + 1, :]
        cols_s.append(lax.dot_general(Wh, ats, (((1,), (1,)), ((), ()))))
        cols_d.append(lax.dot_general(Wh, atd, (((1,), (1,)), ((), ()))))
    vs = jnp.concatenate(cols_s, axis=1)
    vd = jnp.concatenate(cols_d, axis=1)
    as_ref[...] = lax.dot_general(vs, xb, (((0,), (1,)), ((), ())))
    ad_ref[...] = lax.dot_general(vd, xb, (((0,), (1,)), ((), ())))


def _tck1(xp, W, att_src, att_dst):
    return pl.pallas_call(
        _tck1_body,
        grid=(NP // NBLK,),
        in_specs=[
            pl.BlockSpec((NBLK, F_IN), lambda i: (i, 0)),
            pl.BlockSpec((F_IN, H1 * OUT), lambda i: (0, 0)),
            pl.BlockSpec((H1, OUT), lambda i: (0, 0)),
            pl.BlockSpec((H1, OUT), lambda i: (0, 0)),
        ],
        out_specs=[
            pl.BlockSpec((2 * H1, NBLK, HW), lambda i: (0, i, 0)),
            pl.BlockSpec((H1, NBLK), lambda i: (0, i)),
            pl.BlockSpec((H1, NBLK), lambda i: (0, i)),
        ],
        out_shape=[
            jax.ShapeDtypeStruct((2 * H1, NP, HW), jnp.float32),
            jax.ShapeDtypeStruct((H1, NP), jnp.float32),
            jax.ShapeDtypeStruct((H1, NP), jnp.float32),
        ],
    )(xp, W, att_src, att_dst)


def _make_sc_edge(heads_per_core, chunks_per_tile, edge_split):
    cpt = chunks_per_tile
    n_slots = 2 * heads_per_core if not edge_split else 2
    mesh = plsc.VectorSubcoreMesh(core_axis_name="c", subcore_axis_name="s")

    @functools.partial(
        pl.kernel,
        out_type=[
            jax.ShapeDtypeStruct((n_slots * 2 * NP, HW), jnp.float32),
            jax.ShapeDtypeStruct((n_slots * NP,), jnp.float32),
        ],
        mesh=mesh,
        compiler_params=pltpu.CompilerParams(needs_layout_passes=False,
                                             use_tc_tiling_on_sc=False),
        scratch_types=[
            pltpu.VMEM((cpt, CH), jnp.int32),
            pltpu.VMEM((cpt, CH), jnp.int32),
            pltpu.VMEM((NP,), jnp.float32),
            pltpu.VMEM((NP,), jnp.float32),
            [pltpu.VMEM((CH,), jnp.float32) for _ in range(2)],
            [pltpu.VMEM((CH,), jnp.int32) for _ in range(2)],
            [pltpu.VMEM((CH,), jnp.int32) for _ in range(2)],
            [pltpu.VMEM((CH, HW), jnp.float32) for _ in range(2)],
            [pltpu.SemaphoreType.DMA for _ in range(2)],
            pltpu.VMEM((CH, HW), jnp.float32),
            pltpu.VMEM((RPT,), jnp.float32),
            pltpu.VMEM_SHARED((NP, HW), jnp.float32),
            pltpu.VMEM_SHARED((NP,), jnp.float32),
        ],
    )
    def k(h_flat, asrc, adst, src128h, dst128h, acc_out, den_out,
          srcv, dstv, av_s, av_d, exb, sidx, didx, rowbuf, gsem,
          zbuf, zden, accum, densp):
        cid = lax.axis_index("c")
        sid = lax.axis_index("s")
        if edge_split:
            chunk_base = (cid * 16 + sid) * cpt
        else:
            chunk_base = sid * cpt
        pltpu.sync_copy(src128h.at[pl.ds(chunk_base, cpt)], srcv)
        pltpu.sync_copy(dst128h.at[pl.ds(chunk_base, cpt)], dstv)

        def _zrow(i, _):
            for g in range(HW // 16):
                zbuf[i, pl.ds(g * 16, 16)] = jnp.zeros((16,), jnp.float32)
            return 0
        lax.fori_loop(0, CH, _zrow, 0)

        def _zdenb(i, _):
            zden[pl.ds(i * 16, 16)] = jnp.zeros((16,), jnp.float32)
            return 0
        lax.fori_loop(0, RPT // 16, _zdenb, 0)


        def per_pass(pi, _):
            hi = pi // 2
            fh = pi % 2
            if edge_split:
                head = hi
                slot = cid
            else:
                head = cid * heads_per_core + hi
                slot = head
            for j in range(RPT // CH):
                pltpu.sync_copy(
                    zbuf, accum.at[pl.ds(sid * RPT + j * CH, CH)])

            @pl.when(fh == 0)
            def _stage():
                pltpu.sync_copy(zden, densp.at[pl.ds(sid * RPT, RPT)])
                pltpu.sync_copy(asrc.at[pl.ds(head * NP, NP)], av_s)
                pltpu.sync_copy(adst.at[pl.ds(head * NP, NP)], av_d)

            plsc.subcore_barrier()
            hrow_base = (head * 2 + fh) * NP

            def compute_idx(c, s):
                for g in range(CH // 16):
                    s16 = srcv[c, pl.ds(g * 16, 16)]
                    d16 = dstv[c, pl.ds(g * 16, 16)]
                    sa = plsc.load_gather(av_s, [s16])
                    da = plsc.load_gather(av_d, [d16])
                    e = sa + da
                    e = jnp.where(e >= 0.0, e, e * jnp.float32(0.2))
                    ex = jnp.exp(e)
                    exb[s][pl.ds(g * 16, 16)] = ex
                    sidx[s][pl.ds(g * 16, 16)] = s16 + hrow_base
                    didx[s][pl.ds(g * 16, 16)] = d16

            def start_gather(s):
                pltpu.async_copy(h_flat.at[sidx[s]], rowbuf[s], gsem[s])

            def wait_gather(s):
                pltpu.make_async_copy(
                    h_flat.at[sidx[s]], rowbuf[s], gsem[s]).wait()

            def scale(s):
                def quad(q, __):
                    for u in range(8):
                        ei = q * 8 + u
                        w = plsc.load_gather(
                            exb[s], [jnp.full((16,), ei, jnp.int32)])
                        for g in range(HW // 16):
                            rowbuf[s][ei, pl.ds(g * 16, 16)] = (
                                rowbuf[s][ei, pl.ds(g * 16, 16)] * w)
                    return 0
                lax.fori_loop(0, CH // 8, quad, 0)

            compute_idx(0, 0)
            start_gather(0)
            compute_idx(1, 1)
            start_gather(1)

            def per_quad(kq, _):
                for s in range(2):
                    c = kq * 2 + s
                    wait_gather(s)
                    scale(s)
                    pltpu.sync_copy(rowbuf[s], accum.at[didx[s]], add=True)

                    @pl.when(fh == 0)
                    def _den():
                        pltpu.sync_copy(exb[s], densp.at[didx[s]], add=True)

                    @pl.when(c + 2 < cpt)
                    def _next():
                        compute_idx(c + 2, s)
                        start_gather(s)
                return 0
            lax.fori_loop(0, cpt // 2, per_quad, 0)

            plsc.subcore_barrier()
            for j in range(RPT // CH):
                r0 = sid * RPT + j * CH
                pltpu.sync_copy(
                    accum.at[pl.ds(r0, CH)],
                    acc_out.at[pl.ds((slot * 2 + fh) * NP + r0, CH)])

            @pl.when(fh == 1)
            def _wden():
                pltpu.sync_copy(
                    densp.at[pl.ds(sid * RPT, RPT)],
                    den_out.at[pl.ds(slot * NP + sid * RPT, RPT)])

            plsc.subcore_barrier()
            return 0
        lax.fori_loop(0, heads_per_core * 2, per_pass, 0)

    return k


_sck1 = _make_sc_edge(heads_per_core=H1 // 2, chunks_per_tile=NCH // 16,
                      edge_split=False)
_sck2 = _make_sc_edge(heads_per_core=1, chunks_per_tile=NCH // 32,
                      edge_split=True)


def _tck2_body(acc_ref, den_ref, b1_ref, w2_ref, a2s_ref, a2d_ref,
               h2_ref, as_ref, ad_ref):
    W2 = w2_ref[...]
    h2 = jnp.zeros((NBLK, OUT), jnp.float32)
    asum = jnp.zeros((1, NBLK), jnp.float32)
    dsum = jnp.zeros((1, NBLK), jnp.float32)
    for h in range(H1):
        den = den_ref[h]
        r = 1.0 / (den + jnp.float32(1e-16))
        xh = jnp.concatenate([acc_ref[2 * h], acc_ref[2 * h + 1]], axis=1)
        xh = xh * r
        xh = xh + b1_ref[pl.ds(h * OUT, OUT)][None, :]
        xh = jnp.where(xh > 0.0, xh, jnp.exp(xh) - 1.0)
        W2h = W2[h * OUT:(h + 1) * OUT, :]
        h2 = h2 + jnp.dot(xh, W2h, preferred_element_type=jnp.float32)
        v2s = lax.dot_general(W2h, a2s_ref[...], (((1,), (1,)), ((), ())))
        v2d = lax.dot_general(W2h, a2d_ref[...], (((1,), (1,)), ((), ())))
        asum = asum + lax.dot_general(v2s, xh, (((0,), (1,)), ((), ())))
        dsum = dsum + lax.dot_general(v2d, xh, (((0,), (1,)), ((), ())))
    h2_ref[0] = h2[:, :HW]
    h2_ref[1] = h2[:, HW:]
    as_ref[...] = asum
    ad_ref[...] = dsum


def _tck2(acc1, den1, b1, W2, att2_src, att2_dst):
    return pl.pallas_call(
        _tck2_body,
        grid=(NP // NBLK,),
        in_specs=[
            pl.BlockSpec((2 * H1, NBLK, HW), lambda i: (0, i, 0)),
            pl.BlockSpec((H1, NBLK, 1), lambda i: (0, i, 0)),
            pl.BlockSpec((H1 * OUT,), lambda i: (0,)),
            pl.BlockSpec((H1 * OUT, OUT), lambda i: (0, 0)),
            pl.BlockSpec((1, OUT), lambda i: (0, 0)),
            pl.BlockSpec((1, OUT), lambda i: (0, 0)),
        ],
        out_specs=[
            pl.BlockSpec((2, NBLK, HW), lambda i: (0, i, 0)),
            pl.BlockSpec((1, NBLK), lambda i: (0, i)),
            pl.BlockSpec((1, NBLK), lambda i: (0, i)),
        ],
        out_shape=[
            jax.ShapeDtypeStruct((2, NP, HW), jnp.float32),
            jax.ShapeDtypeStruct((1, NP), jnp.float32),
            jax.ShapeDtypeStruct((1, NP), jnp.float32),
        ],
    )(acc1, den1, b1, W2, att2_src, att2_dst)


def _tck3_body(acc_ref, den_ref, b2_ref, seg_ref, fcw_ref, fcb_ref, out_ref):
    lo = acc_ref[0] + acc_ref[2]
    hi = acc_ref[1] + acc_ref[3]
    hb = jnp.concatenate([lo, hi], axis=1)
    dn = den_ref[0] + den_ref[1]
    hb = hb / (dn + jnp.float32(1e-16))
    hb = hb + b2_ref[...][None, :]
    hb = jnp.where(hb > 0.0, hb, jnp.exp(hb) - 1.0)
    seg = seg_ref[...].reshape(NBLK, 1)
    neg = jnp.float32(-3.0e38)
    rows = []
    for g in range(G):
        masked = jnp.where(seg == g, hb, neg)
        rows.append(jnp.max(masked, axis=0, keepdims=True))
    gmax = jnp.concatenate(rows, axis=0)

    @pl.when(pl.program_id(0) == 0)
    def _init():
        out_ref[...] = jnp.full_like(out_ref, neg)

    out_ref[...] = jnp.maximum(out_ref[...], gmax)

    @pl.when(pl.program_id(0) == pl.num_programs(0) - 1)
    def _fin():
        g = out_ref[...]
        g = jnp.where(g > jnp.float32(-1.0e38), g, 0.0)
        out_ref[...] = jnp.maximum(
            jnp.dot(g, fcw_ref[...], preferred_element_type=jnp.float32)
            + fcb_ref[...][None, :], 0.0)


def _tck3(acc2, den2, b2, segp, fc_W, fc_b):
    return pl.pallas_call(
        _tck3_body,
        grid=(NP // NBLK,),
        in_specs=[
            pl.BlockSpec((4, NBLK, HW), lambda i: (0, i, 0)),
            pl.BlockSpec((2, NBLK, 1), lambda i: (0, i, 0)),
            pl.BlockSpec((OUT,), lambda i: (0,)),
            pl.BlockSpec((1, 1, NBLK), lambda i: (i, 0, 0)),
            pl.BlockSpec((OUT, OUT), lambda i: (0, 0)),
            pl.BlockSpec((OUT,), lambda i: (0,)),
        ],
        out_specs=pl.BlockSpec((G, OUT), lambda i: (0, 0)),
        out_shape=jax.ShapeDtypeStruct((G, OUT), jnp.float32),
    )(acc2, den2, b2, segp, fc_W, fc_b)


def kernel(x, edge_index, batch, gat1_W, gat1_att_src, gat1_att_dst, gat1_b,
           gat2_W, gat2_att_src, gat2_att_dst, gat2_b, fc_W, fc_b):
    xp = jnp.pad(x, ((0, NP - N), (0, 0)))
    ei = jnp.pad(edge_index, ((0, 0), (0, EP - E)), constant_values=N)
    src128 = ei[0].reshape(NCH, CH)
    dst128 = ei[1].reshape(NCH, CH)

    h1, as1, ad1 = _tck1(xp, gat1_W, gat1_att_src, gat1_att_dst)
    acc1, den1 = _sck1(h1.reshape(2 * H1 * NP, HW), as1.reshape(H1 * NP),
                       ad1.reshape(H1 * NP), src128, dst128)
    h2, a2s, a2d = _tck2(acc1.reshape(2 * H1, NP, HW),
                         den1.reshape(H1, NP, 1),
                         gat1_b, gat2_W, gat2_att_src, gat2_att_dst)
    acc2, den2 = _sck2(h2.reshape(2 * NP, HW), a2s.reshape(NP),
                       a2d.reshape(NP), src128, dst128)

    segp = jnp.pad(batch, (0, NP - N), constant_values=100)
    segp = segp.reshape(NP // NBLK, 1, NBLK).astype(jnp.int32)
    return _tck3(acc2.reshape(4, NP, HW), den2.reshape(2, NP, 1),
                 gat2_b, segp, fc_W, fc_b)


# speedup vs baseline: 1.0069x; 1.0069x over previous
import functools

import jax
import jax.numpy as jnp
from jax import lax
from jax.experimental import pallas as pl
from jax.experimental.pallas import tpu as pltpu
from jax.experimental.pallas import tpu_sc as plsc

N = 10000
E = 320000
F_IN = 128
OUT = 128
HW = 64
H1 = 10
G = 64

NP = 10240
NBLK = 1024
EP = 327680
CH = 128
NCH = EP // CH
RPT = NP // 16


def _tck1_body(x_ref, w_ref, atts_ref, attd_ref, h_ref, as_ref, ad_ref):
    xb = x_ref[...]
    W = w_ref[...]
    hb = jnp.dot(xb, W, preferred_element_type=jnp.float32)
    cols_s = []
    cols_d = []
    for h in range(H1):
        h_ref[2 * h] = hb[:, h * OUT:h * OUT + HW]
        h_ref[2 * h + 1] = hb[:, h * OUT + HW:(h + 1) * OUT]
        Wh = W[:, h * OUT:(h + 1) * OUT]
        ats = atts_ref[...][h:h + 1, :]
        atd = attd_ref[...][h:h + 1, :]
        cols_s.append(lax.dot_general(Wh, ats, (((1,), (1,)), ((), ()))))
        cols_d.append(lax.dot_general(Wh, atd, (((1,), (1,)), ((), ()))))
    vs = jnp.concatenate(cols_s, axis=1)
    vd = jnp.concatenate(cols_d, axis=1)
    as_ref[...] = lax.dot_general(vs, xb, (((0,), (1,)), ((), ())))
    ad_ref[...] = lax.dot_general(vd, xb, (((0,), (1,)), ((), ())))


def _tck1(xp, W, att_src, att_dst):
    return pl.pallas_call(
        _tck1_body,
        grid=(NP // NBLK,),
        in_specs=[
            pl.BlockSpec((NBLK, F_IN), lambda i: (i, 0)),
            pl.BlockSpec((F_IN, H1 * OUT), lambda i: (0, 0)),
            pl.BlockSpec((H1, OUT), lambda i: (0, 0)),
            pl.BlockSpec((H1, OUT), lambda i: (0, 0)),
        ],
        out_specs=[
            pl.BlockSpec((2 * H1, NBLK, HW), lambda i: (0, i, 0)),
            pl.BlockSpec((H1, NBLK), lambda i: (0, i)),
            pl.BlockSpec((H1, NBLK), lambda i: (0, i)),
        ],
        out_shape=[
            jax.ShapeDtypeStruct((2 * H1, NP, HW), jnp.float32),
            jax.ShapeDtypeStruct((H1, NP), jnp.float32),
            jax.ShapeDtypeStruct((H1, NP), jnp.float32),
        ],
    )(xp, W, att_src, att_dst)


def _make_sc_edge(heads_per_core, chunks_per_tile, edge_split):
    cpt = chunks_per_tile
    n_slots = 2 * heads_per_core if not edge_split else 2
    mesh = plsc.VectorSubcoreMesh(core_axis_name="c", subcore_axis_name="s")

    @functools.partial(
        pl.kernel,
        out_type=[
            jax.ShapeDtypeStruct((n_slots * 2 * NP, HW), jnp.float32),
            jax.ShapeDtypeStruct((n_slots * NP,), jnp.float32),
        ],
        mesh=mesh,
        compiler_params=pltpu.CompilerParams(needs_layout_passes=False,
                                             use_tc_tiling_on_sc=False),
        scratch_types=[
            pltpu.VMEM((cpt, CH), jnp.int32),
            pltpu.VMEM((cpt, CH), jnp.int32),
            pltpu.VMEM((NP,), jnp.float32),
            pltpu.VMEM((NP,), jnp.float32),
            [pltpu.VMEM((CH,), jnp.float32) for _ in range(2)],
            [pltpu.VMEM((CH,), jnp.int32) for _ in range(2)],
            [pltpu.VMEM((CH,), jnp.int32) for _ in range(2)],
            [pltpu.VMEM((CH, HW), jnp.float32) for _ in range(2)],
            [pltpu.SemaphoreType.DMA for _ in range(2)],
            pltpu.VMEM((CH, HW), jnp.float32),
            pltpu.VMEM((RPT,), jnp.float32),
            pltpu.VMEM_SHARED((NP, HW), jnp.float32),
            pltpu.VMEM_SHARED((NP,), jnp.float32),
        ],
    )
    def k(h_flat, asrc, adst, src128h, dst128h, acc_out, den_out,
          srcv, dstv, av_s, av_d, exb, sidx, didx, rowbuf, gsem,
          zbuf, zden, accum, densp):
        cid = lax.axis_index("c")
        sid = lax.axis_index("s")
        if edge_split:
            chunk_base = (cid * 16 + sid) * cpt
        else:
            chunk_base = sid * cpt
        pltpu.sync_copy(src128h.at[pl.ds(chunk_base, cpt)], srcv)
        pltpu.sync_copy(dst128h.at[pl.ds(chunk_base, cpt)], dstv)

        def _zrow(i, _):
            for g in range(HW // 16):
                zbuf[i, pl.ds(g * 16, 16)] = jnp.zeros((16,), jnp.float32)
            return 0
        lax.fori_loop(0, CH, _zrow, 0)

        def _zdenb(i, _):
            zden[pl.ds(i * 16, 16)] = jnp.zeros((16,), jnp.float32)
            return 0
        lax.fori_loop(0, RPT // 16, _zdenb, 0)


        def per_pass(pi, _):
            hi = pi // 2
            fh = pi % 2
            if edge_split:
                head = hi
                slot = cid
            else:
                head = cid * heads_per_core + hi
                slot = head
            for j in range(RPT // CH):
                pltpu.sync_copy(
                    zbuf, accum.at[pl.ds(sid * RPT + j * CH, CH)])

            @pl.when(fh == 0)
            def _stage():
                pltpu.sync_copy(zden, densp.at[pl.ds(sid * RPT, RPT)])
                pltpu.sync_copy(asrc.at[pl.ds(head * NP, NP)], av_s)
                pltpu.sync_copy(adst.at[pl.ds(head * NP, NP)], av_d)

            plsc.subcore_barrier()
            hrow_base = (head * 2 + fh) * NP

            def compute_idx(c, s):
                for g in range(CH // 16):
                    s16 = srcv[c, pl.ds(g * 16, 16)]
                    d16 = dstv[c, pl.ds(g * 16, 16)]
                    sa = plsc.load_gather(av_s, [s16])
                    da = plsc.load_gather(av_d, [d16])
                    e = sa + da
                    e = jnp.where(e >= 0.0, e, e * jnp.float32(0.2))
                    ex = jnp.exp(e)
                    exb[s][pl.ds(g * 16, 16)] = ex
                    sidx[s][pl.ds(g * 16, 16)] = s16 + hrow_base
                    didx[s][pl.ds(g * 16, 16)] = d16

            def start_gather(s):
                pltpu.async_copy(h_flat.at[sidx[s]], rowbuf[s], gsem[s])

            def wait_gather(s):
                pltpu.make_async_copy(
                    h_flat.at[sidx[s]], rowbuf[s], gsem[s]).wait()

            def scale(s):
                def quad(q, __):
                    for u in range(4):
                        ei = q * 4 + u
                        w = plsc.load_gather(
                            exb[s], [jnp.full((16,), ei, jnp.int32)])
                        for g in range(HW // 16):
                            rowbuf[s][ei, pl.ds(g * 16, 16)] = (
                                rowbuf[s][ei, pl.ds(g * 16, 16)] * w)
                    return 0
                lax.fori_loop(0, CH // 4, quad, 0)

            compute_idx(0, 0)
            start_gather(0)
            compute_idx(1, 1)
            start_gather(1)

            def per_quad(kq, _):
                for s in range(2):
                    c = kq * 2 + s
                    wait_gather(s)
                    scale(s)
                    pltpu.sync_copy(rowbuf[s], accum.at[didx[s]], add=True)

                    @pl.when(fh == 0)
                    def _den():
                        pltpu.sync_copy(exb[s], densp.at[didx[s]], add=True)

                    @pl.when(c + 2 < cpt)
                    def _next():
                        compute_idx(c + 2, s)
                        start_gather(s)
                return 0
            lax.fori_loop(0, cpt // 2, per_quad, 0)

            plsc.subcore_barrier()
            for j in range(RPT // CH):
                r0 = sid * RPT + j * CH
                pltpu.sync_copy(
                    accum.at[pl.ds(r0, CH)],
                    acc_out.at[pl.ds((slot * 2 + fh) * NP + r0, CH)])

            @pl.when(fh == 1)
            def _wden():
                pltpu.sync_copy(
                    densp.at[pl.ds(sid * RPT, RPT)],
                    den_out.at[pl.ds(slot * NP + sid * RPT, RPT)])

            plsc.subcore_barrier()
            return 0
        lax.fori_loop(0, heads_per_core * 2, per_pass, 0)

    return k


_sck1 = _make_sc_edge(heads_per_core=H1 // 2, chunks_per_tile=NCH // 16,
                      edge_split=False)
_sck2 = _make_sc_edge(heads_per_core=1, chunks_per_tile=NCH // 32,
                      edge_split=True)


def _tck2_body(acc_ref, den_ref, b1_ref, w2_ref, a2s_ref, a2d_ref,
               h2_ref, as_ref, ad_ref):
    W2 = w2_ref[...]
    h2 = jnp.zeros((NBLK, OUT), jnp.float32)
    asum = jnp.zeros((1, NBLK), jnp.float32)
    dsum = jnp.zeros((1, NBLK), jnp.float32)
    for h in range(H1):
        den = den_ref[h]
        r = 1.0 / (den + jnp.float32(1e-16))
        xh = jnp.concatenate([acc_ref[2 * h], acc_ref[2 * h + 1]], axis=1)
        xh = xh * r
        xh = xh + b1_ref[pl.ds(h * OUT, OUT)][None, :]
        xh = jnp.where(xh > 0.0, xh, jnp.exp(xh) - 1.0)
        W2h = W2[h * OUT:(h + 1) * OUT, :]
        h2 = h2 + jnp.dot(xh, W2h, preferred_element_type=jnp.float32)
        v2s = lax.dot_general(W2h, a2s_ref[...], (((1,), (1,)), ((), ())))
        v2d = lax.dot_general(W2h, a2d_ref[...], (((1,), (1,)), ((), ())))
        asum = asum + lax.dot_general(v2s, xh, (((0,), (1,)), ((), ())))
        dsum = dsum + lax.dot_general(v2d, xh, (((0,), (1,)), ((), ())))
    h2_ref[0] = h2[:, :HW]
    h2_ref[1] = h2[:, HW:]
    as_ref[...] = asum
    ad_ref[...] = dsum


def _tck2(acc1, den1, b1, W2, att2_src, att2_dst):
    return pl.pallas_call(
        _tck2_body,
        grid=(NP // NBLK,),
        in_specs=[
            pl.BlockSpec((2 * H1, NBLK, HW), lambda i: (0, i, 0)),
            pl.BlockSpec((H1, NBLK, 1), lambda i: (0, i, 0)),
            pl.BlockSpec((H1 * OUT,), lambda i: (0,)),
            pl.BlockSpec((H1 * OUT, OUT), lambda i: (0, 0)),
            pl.BlockSpec((1, OUT), lambda i: (0, 0)),
            pl.BlockSpec((1, OUT), lambda i: (0, 0)),
        ],
        out_specs=[
            pl.BlockSpec((2, NBLK, HW), lambda i: (0, i, 0)),
            pl.BlockSpec((1, NBLK), lambda i: (0, i)),
            pl.BlockSpec((1, NBLK), lambda i: (0, i)),
        ],
        out_shape=[
            jax.ShapeDtypeStruct((2, NP, HW), jnp.float32),
            jax.ShapeDtypeStruct((1, NP), jnp.float32),
            jax.ShapeDtypeStruct((1, NP), jnp.float32),
        ],
    )(acc1, den1, b1, W2, att2_src, att2_dst)


def _tck3_body(acc_ref, den_ref, b2_ref, seg_ref, fcw_ref, fcb_ref, out_ref):
    lo = acc_ref[0] + acc_ref[2]
    hi = acc_ref[1] + acc_ref[3]
    hb = jnp.concatenate([lo, hi], axis=1)
    dn = den_ref[0] + den_ref[1]
    hb = hb / (dn + jnp.float32(1e-16))
    hb = hb + b2_ref[...][None, :]
    hb = jnp.where(hb > 0.0, hb, jnp.exp(hb) - 1.0)
    seg = seg_ref[...].reshape(NBLK, 1)
    neg = jnp.float32(-3.0e38)
    rows = []
    for g in range(G):
        masked = jnp.where(seg == g, hb, neg)
        rows.append(jnp.max(masked, axis=0, keepdims=True))
    gmax = jnp.concatenate(rows, axis=0)

    @pl.when(pl.program_id(0) == 0)
    def _init():
        out_ref[...] = jnp.full_like(out_ref, neg)

    out_ref[...] = jnp.maximum(out_ref[...], gmax)

    @pl.when(pl.program_id(0) == pl.num_programs(0) - 1)
    def _fin():
        g = out_ref[...]
        g = jnp.where(g > jnp.float32(-1.0e38), g, 0.0)
        out_ref[...] = jnp.maximum(
            jnp.dot(g, fcw_ref[...], preferred_element_type=jnp.float32)
            + fcb_ref[...][None, :], 0.0)


def _tck3(acc2, den2, b2, segp, fc_W, fc_b):
    return pl.pallas_call(
        _tck3_body,
        grid=(NP // NBLK,),
        in_specs=[
            pl.BlockSpec((4, NBLK, HW), lambda i: (0, i, 0)),
            pl.BlockSpec((2, NBLK, 1), lambda i: (0, i, 0)),
            pl.BlockSpec((OUT,), lambda i: (0,)),
            pl.BlockSpec((1, 1, NBLK), lambda i: (i, 0, 0)),
            pl.BlockSpec((OUT, OUT), lambda i: (0, 0)),
            pl.BlockSpec((OUT,), lambda i: (0,)),
        ],
        out_specs=pl.BlockSpec((G, OUT), lambda i: (0, 0)),
        out_shape=jax.ShapeDtypeStruct((G, OUT), jnp.float32),
    )(acc2, den2, b2, segp, fc_W, fc_b)


def kernel(x, edge_index, batch, gat1_W, gat1_att_src, gat1_att_dst, gat1_b,
           gat2_W, gat2_att_src, gat2_att_dst, gat2_b, fc_W, fc_b):
    xp = jnp.pad(x, ((0, NP - N), (0, 0)))
    ei = jnp.pad(edge_index, ((0, 0), (0, EP - E)), constant_values=N)
    src128 = ei[0].reshape(NCH, CH)
    dst128 = ei[1].reshape(NCH, CH)

    h1, as1, ad1 = _tck1(xp, gat1_W, gat1_att_src, gat1_att_dst)
    acc1, den1 = _sck1(h1.reshape(2 * H1 * NP, HW), as1.reshape(H1 * NP),
                       ad1.reshape(H1 * NP), src128, dst128)
    h2, a2s, a2d = _tck2(acc1.reshape(2 * H1, NP, HW),
                         den1.reshape(H1, NP, 1),
                         gat1_b, gat2_W, gat2_att_src, gat2_att_dst)
    acc2, den2 = _sck2(h2.reshape(2 * NP, HW), a2s.reshape(NP),
                       a2d.reshape(NP), src128, dst128)

    segp = jnp.pad(batch, (0, NP - N), constant_values=100)
    segp = segp.reshape(NP // NBLK, 1, NBLK).astype(jnp.int32)
    return _tck3(acc2.reshape(4, NP, HW), den2.reshape(2, NP, 1),
                 gat2_b, segp, fc_W, fc_b)
